# Initial kernel scaffold; baseline (speedup 1.0000x reference)
import jax
import jax.numpy as jnp
from jax.experimental import pallas as pl


def kernel(points1, points2):
    raise NotImplementedError("write your pallas kernel here")


# speedup vs baseline: 1.3858x; 1.3858x over previous
import jax
import jax.numpy as jnp
from jax.experimental import pallas as pl
from jax.experimental.pallas import tpu as pltpu

N_BLK = 256


def _chamfer_body(p1_ref, p2t_ref, d1_ref, i1_ref, d2_ref, i2_ref):
    n = pl.program_id(1)
    n_blk = d1_ref.shape[1]
    big_m = jnp.int32(p2t_ref.shape[2])
    big_n = jnp.int32(n_blk * pl.num_programs(1))

    p1 = p1_ref[0]
    p2t = p2t_ref[0]
    a_x = p1[:, 0:1]
    a_y = p1[:, 1:2]
    a_z = p1[:, 2:3]
    b_x = p2t[0:1, :]
    b_y = p2t[1:2, :]
    b_z = p2t[2:3, :]

    ab_x = a_x.astype(jnp.bfloat16).astype(jnp.float32)
    ab_y = a_y.astype(jnp.bfloat16).astype(jnp.float32)
    ab_z = a_z.astype(jnp.bfloat16).astype(jnp.float32)
    bb_x = b_x.astype(jnp.bfloat16).astype(jnp.float32)
    bb_y = b_y.astype(jnp.bfloat16).astype(jnp.float32)
    bb_z = b_z.astype(jnp.bfloat16).astype(jnp.float32)
    cross = ab_x * bb_x + ab_y * bb_y + ab_z * bb_z
    x1sq = a_x * a_x + a_y * a_y + a_z * a_z
    x2sq = b_x * b_x + b_y * b_y + b_z * b_z
    d = (x1sq + x2sq) - 2.0 * cross

    dmin = jnp.min(d, axis=1, keepdims=True)
    iota_m = jax.lax.broadcasted_iota(jnp.int32, d.shape, 1)
    imin = jnp.min(jnp.where(d == dmin, iota_m, big_m), axis=1, keepdims=True)
    d1_ref[0] = dmin
    i1_ref[0] = imin

    cmin = jnp.min(d, axis=0, keepdims=True)
    iota_n = jax.lax.broadcasted_iota(jnp.int32, d.shape, 0)
    cidx = jnp.min(jnp.where(d == cmin, iota_n, big_n), axis=0, keepdims=True)
    cidx = cidx + n * n_blk

    @pl.when(n == 0)
    def _():
        d2_ref[0] = cmin
        i2_ref[0] = cidx

    @pl.when(n != 0)
    def _():
        prev_d = d2_ref[0]
        prev_i = i2_ref[0]
        take_new = cmin < prev_d
        d2_ref[0] = jnp.where(take_new, cmin, prev_d)
        i2_ref[0] = jnp.where(take_new, cidx, prev_i)


def kernel(points1, points2):
    B, N, D = points1.shape
    M = points2.shape[1]
    p2t = points2.transpose(0, 2, 1)

    d1, i1, d2, i2 = pl.pallas_call(
        _chamfer_body,
        grid=(B, N // N_BLK),
        in_specs=[
            pl.BlockSpec((1, N_BLK, D), lambda b, n: (b, n, 0)),
            pl.BlockSpec((1, D, M), lambda b, n: (b, 0, 0)),
        ],
        out_specs=[
            pl.BlockSpec((1, N_BLK, 1), lambda b, n: (b, n, 0)),
            pl.BlockSpec((1, N_BLK, 1), lambda b, n: (b, n, 0)),
            pl.BlockSpec((1, 1, M), lambda b, n: (b, 0, 0)),
            pl.BlockSpec((1, 1, M), lambda b, n: (b, 0, 0)),
        ],
        out_shape=[
            jax.ShapeDtypeStruct((B, N, 1), jnp.float32),
            jax.ShapeDtypeStruct((B, N, 1), jnp.int32),
            jax.ShapeDtypeStruct((B, 1, M), jnp.float32),
            jax.ShapeDtypeStruct((B, 1, M), jnp.int32),
        ],
    )(points1, p2t)

    return (i1[..., 0], i2[:, 0, :], d1[..., 0], d2[:, 0, :])


# speedup vs baseline: 1.5041x; 1.0854x over previous
import jax
import jax.numpy as jnp
from jax.experimental import pallas as pl
from jax.experimental.pallas import tpu as pltpu

N_BLK = 256
K_PAD = 8


def _chamfer_body(p1_ref, p2t_ref, p1b_ref, p2tb_ref,
                  d1_ref, i1_ref, d2_ref, i2_ref):
    n = pl.program_id(1)
    n_blk = d1_ref.shape[1]
    big_m = jnp.int32(p2t_ref.shape[2])
    big_n = jnp.int32(n_blk * pl.num_programs(1))

    p1 = p1_ref[0]
    p2t = p2t_ref[0]
    a_x = p1[:, 0:1]
    a_y = p1[:, 1:2]
    a_z = p1[:, 2:3]
    b_x = p2t[0:1, :]
    b_y = p2t[1:2, :]
    b_z = p2t[2:3, :]

    cross = jax.lax.dot_general(
        p1b_ref[0], p2tb_ref[0],
        dimension_numbers=(((1,), (0,)), ((), ())),
        preferred_element_type=jnp.float32,
    )
    x1sq = a_x * a_x + a_y * a_y + a_z * a_z
    x2sq = b_x * b_x + b_y * b_y + b_z * b_z
    d = (x1sq + x2sq) - 2.0 * cross

    dmin = jnp.min(d, axis=1, keepdims=True)
    iota_m = jax.lax.broadcasted_iota(jnp.int32, d.shape, 1)
    imin = jnp.min(jnp.where(d == dmin, iota_m, big_m), axis=1, keepdims=True)
    d1_ref[0] = dmin
    i1_ref[0] = imin

    cmin = jnp.min(d, axis=0, keepdims=True)
    iota_n = jax.lax.broadcasted_iota(jnp.int32, d.shape, 0)
    cidx = jnp.min(jnp.where(d == cmin, iota_n, big_n), axis=0, keepdims=True)
    cidx = cidx + n * n_blk

    @pl.when(n == 0)
    def _():
        d2_ref[0] = cmin
        i2_ref[0] = cidx

    @pl.when(n != 0)
    def _():
        prev_d = d2_ref[0]
        prev_i = i2_ref[0]
        take_new = cmin < prev_d
        d2_ref[0] = jnp.where(take_new, cmin, prev_d)
        i2_ref[0] = jnp.where(take_new, cidx, prev_i)


def kernel(points1, points2):
    B, N, D = points1.shape
    M = points2.shape[1]
    p2t = points2.transpose(0, 2, 1)

    pad = [(0, 0), (0, 0), (0, K_PAD - D)]
    p1b = jnp.pad(points1.astype(jnp.bfloat16), pad)
    p2tb = jnp.pad(p2t.astype(jnp.bfloat16),
                   [(0, 0), (0, K_PAD - D), (0, 0)])

    d1, i1, d2, i2 = pl.pallas_call(
        _chamfer_body,
        grid=(B, N // N_BLK),
        in_specs=[
            pl.BlockSpec((1, N_BLK, D), lambda b, n: (b, n, 0)),
            pl.BlockSpec((1, D, M), lambda b, n: (b, 0, 0)),
            pl.BlockSpec((1, N_BLK, K_PAD), lambda b, n: (b, n, 0)),
            pl.BlockSpec((1, K_PAD, M), lambda b, n: (b, 0, 0)),
        ],
        out_specs=[
            pl.BlockSpec((1, N_BLK, 1), lambda b, n: (b, n, 0)),
            pl.BlockSpec((1, N_BLK, 1), lambda b, n: (b, n, 0)),
            pl.BlockSpec((1, 1, M), lambda b, n: (b, 0, 0)),
            pl.BlockSpec((1, 1, M), lambda b, n: (b, 0, 0)),
        ],
        out_shape=[
            jax.ShapeDtypeStruct((B, N, 1), jnp.float32),
            jax.ShapeDtypeStruct((B, N, 1), jnp.int32),
            jax.ShapeDtypeStruct((B, 1, M), jnp.float32),
            jax.ShapeDtypeStruct((B, 1, M), jnp.int32),
        ],
    )(points1, p2t, p1b, p2tb)

    return (i1[..., 0], i2[:, 0, :], d1[..., 0], d2[:, 0, :])


# speedup vs baseline: 1.9494x; 1.2961x over previous
import jax
import jax.numpy as jnp
from jax.experimental import pallas as pl
from jax.experimental.pallas import tpu as pltpu

N_BLK = 1024
K_PAD = 8


def _chamfer_body(p1_ref, p2t_ref, p1b_ref, p2tb_ref, iota_n_ref, iota_m_ref,
                  d1_ref, i1_ref, d2_ref, i2_ref):
    n = pl.program_id(1)
    big_m = jnp.float32(p2t_ref.shape[2])
    big_n = jnp.float32(iota_n_ref.shape[1] * pl.num_programs(1))

    p1 = p1_ref[0]
    p2t = p2t_ref[0]
    a_x = p1[:, 0:1]
    a_y = p1[:, 1:2]
    a_z = p1[:, 2:3]
    b_x = p2t[0:1, :]
    b_y = p2t[1:2, :]
    b_z = p2t[2:3, :]

    neg2cross = jax.lax.dot_general(
        p1b_ref[0], p2tb_ref[0],
        dimension_numbers=(((1,), (0,)), ((), ())),
        preferred_element_type=jnp.float32,
    )
    x1sq = a_x * a_x + a_y * a_y + a_z * a_z
    x2sq = b_x * b_x + b_y * b_y + b_z * b_z
    d = (x1sq + x2sq) + neg2cross

    iota_n = iota_n_ref[0]
    iota_m = iota_m_ref[0]

    dmin = jnp.min(d, axis=1, keepdims=True)
    imin = jnp.min(jnp.where(d == dmin, iota_m, big_m), axis=1, keepdims=True)
    d1_ref[0] = dmin
    i1_ref[0] = imin.astype(jnp.int32)

    cmin = jnp.min(d, axis=0, keepdims=True)
    cidx = jnp.min(jnp.where(d == cmin, iota_n, big_n), axis=0, keepdims=True)

    @pl.when(n == 0)
    def _():
        d2_ref[0] = cmin
        i2_ref[0] = cidx.astype(jnp.int32)

    @pl.when(n != 0)
    def _():
        prev_d = d2_ref[0]
        take_new = cmin < prev_d
        d2_ref[0] = jnp.where(take_new, cmin, prev_d)
        i2_ref[0] = jnp.where(take_new, cidx.astype(jnp.int32), i2_ref[0])


def kernel(points1, points2):
    B, N, D = points1.shape
    M = points2.shape[1]
    p2t = points2.transpose(0, 2, 1)

    pad = [(0, 0), (0, 0), (0, K_PAD - D)]
    p1b = jnp.pad((-2.0 * points1).astype(jnp.bfloat16), pad)
    p2tb = jnp.pad(p2t.astype(jnp.bfloat16),
                   [(0, 0), (0, K_PAD - D), (0, 0)])
    iota_n = jnp.arange(N, dtype=jnp.float32).reshape(1, N, 1)
    iota_m = jnp.arange(M, dtype=jnp.float32).reshape(1, 1, M)

    d1, i1, d2, i2 = pl.pallas_call(
        _chamfer_body,
        grid=(B, N // N_BLK),
        in_specs=[
            pl.BlockSpec((1, N_BLK, D), lambda b, n: (b, n, 0)),
            pl.BlockSpec((1, D, M), lambda b, n: (b, 0, 0)),
            pl.BlockSpec((1, N_BLK, K_PAD), lambda b, n: (b, n, 0)),
            pl.BlockSpec((1, K_PAD, M), lambda b, n: (b, 0, 0)),
            pl.BlockSpec((1, N_BLK, 1), lambda b, n: (0, n, 0)),
            pl.BlockSpec((1, 1, M), lambda b, n: (0, 0, 0)),
        ],
        out_specs=[
            pl.BlockSpec((1, N_BLK, 1), lambda b, n: (b, n, 0)),
            pl.BlockSpec((1, N_BLK, 1), lambda b, n: (b, n, 0)),
            pl.BlockSpec((1, 1, M), lambda b, n: (b, 0, 0)),
            pl.BlockSpec((1, 1, M), lambda b, n: (b, 0, 0)),
        ],
        out_shape=[
            jax.ShapeDtypeStruct((B, N, 1), jnp.float32),
            jax.ShapeDtypeStruct((B, N, 1), jnp.int32),
            jax.ShapeDtypeStruct((B, 1, M), jnp.float32),
            jax.ShapeDtypeStruct((B, 1, M), jnp.int32),
        ],
    )(points1, p2t, p1b, p2tb, iota_n, iota_m)

    return (i1[..., 0], i2[:, 0, :], d1[..., 0], d2[:, 0, :])
